# Initial kernel scaffold; baseline (speedup 1.0000x reference)
import jax
import jax.numpy as jnp
from jax.experimental import pallas as pl


def kernel(states, prior_states, sigma, onsets, init_ss, t_obs, s_obs, noise_scale, correct_prior, correct_lik, forget_lik):
    raise NotImplementedError("write your pallas kernel here")


# speedup vs baseline: 5.7822x; 5.7822x over previous
import functools

import jax
import jax.numpy as jnp
import numpy as np
from jax import lax
from jax.experimental import pallas as pl
from jax.experimental.pallas import tpu as pltpu
from jax.experimental.pallas import tpu_sc as plsc

_N = 262144
_LOG2N = 18
_D = 16
_B = 100
_EPS = 1e-6
_HALF_LOG_2PI = 0.9189385332046727
_WINDOW = 21.5
_TINY = float(np.finfo(np.float32).tiny)
_NEG = np.float32(-1e38)


def _threefry_bits(hi, lo):
    ks0 = np.uint32(0)
    ks1 = np.uint32(7)
    ks2 = np.uint32(0 ^ 7 ^ 0x1BD11BDA)
    x0 = hi + ks0
    x1 = lo + ks1
    ks = (ks0, ks1, ks2)
    rots = ((13, 15, 26, 6), (17, 29, 16, 24))
    for i in range(5):
        for r in rots[i % 2]:
            x0 = x0 + x1
            x1 = (x1 << np.uint32(r)) | (x1 >> np.uint32(32 - r))
            x1 = x1 ^ x0
        x0 = x0 + ks[(i + 1) % 3]
        x1 = x1 + ks[(i + 2) % 3] + np.uint32(i + 1)
    return x0 ^ x1


def _gumbel_from_bits(bits):
    fb = (bits >> np.uint32(9)) | np.uint32(0x3F800000)
    u0 = lax.bitcast_convert_type(fb, jnp.float32) - np.float32(1.0)
    u = jnp.maximum(np.float32(_TINY), u0 + np.float32(_TINY))
    return -jnp.log(-jnp.log(u))


def _softplus(x):
    return jnp.maximum(x, 0.0) + jnp.log1p(jnp.exp(-jnp.abs(x)))


def _fwd_body(par_ref, tobs_ref, sobs_ref, wts_ref, st_ref, pr_ref, nz_ref,
              on_ref, is_ref, s2_ref, coef_ref, ns_ref, lw_ref):
    ns = st_ref[...] + nz_ref[...] * s2_ref[...]
    ns_ref[...] = ns
    rate = _softplus(ns[0]) + np.float32(_EPS)
    disp = _softplus(ns[1]) + np.float32(_EPS)
    inv_r = 1.0 / rate
    inv_d = 1.0 / disp
    ldc = jnp.log(disp) + np.float32(_HALF_LOG_2PI)
    on = on_ref[...]
    iss = is_ref[...]
    cb = on.shape[0]

    def bbody(b, acc):
        tb = tobs_ref[b]
        sb = sobs_ref[b]
        wb = wts_ref[b]
        z = ((tb - on) - (sb - iss) * inv_r) * inv_d
        return acc + wb * (np.float32(-0.5) * (z * z) - ldc)

    acc = lax.fori_loop(0, _B, bbody, jnp.zeros((cb,), jnp.float32))
    diff = ns - pr_ref[...]
    accp = jnp.sum(coef_ref[...] * (diff * diff), axis=0)
    lw_ref[...] = par_ref[0] * acc + accp


def _forward(par, t_obs, s_obs, wts, st_t, pr_t, nz_t, onsets, init_ss, s2,
             coef, n, interpret=False):
    cb = 8192
    grid = (n // cb,)
    return pl.pallas_call(
        _fwd_body,
        grid=grid,
        in_specs=[
            pl.BlockSpec(memory_space=pltpu.SMEM),
            pl.BlockSpec(memory_space=pltpu.SMEM),
            pl.BlockSpec(memory_space=pltpu.SMEM),
            pl.BlockSpec(memory_space=pltpu.SMEM),
            pl.BlockSpec((_D, cb), lambda i: (0, i)),
            pl.BlockSpec((_D, cb), lambda i: (0, i)),
            pl.BlockSpec((_D, cb), lambda i: (0, i)),
            pl.BlockSpec((cb,), lambda i: (i,)),
            pl.BlockSpec((cb,), lambda i: (i,)),
            pl.BlockSpec((_D, 1), lambda i: (0, 0)),
            pl.BlockSpec((_D, 1), lambda i: (0, 0)),
        ],
        out_specs=[
            pl.BlockSpec((_D, cb), lambda i: (0, i)),
            pl.BlockSpec((cb,), lambda i: (i,)),
        ],
        out_shape=[
            jax.ShapeDtypeStruct((_D, n), jnp.float32),
            jax.ShapeDtypeStruct((n,), jnp.float32),
        ],
        interpret=interpret,
    )(par, t_obs, s_obs, wts, st_t, pr_t, nz_t, onsets, init_ss, s2, coef)


def _softmax_body(lw_ref, w_ref):
    lw = lw_ref[...]
    e = jnp.exp(lw - jnp.max(lw))
    w_ref[...] = e / jnp.sum(e)


def _softmax(log_w, n, interpret=False):
    return pl.pallas_call(
        _softmax_body,
        out_shape=jax.ShapeDtypeStruct((n,), jnp.float32),
        interpret=interpret,
    )(log_w)


_CHUNK = 2048


def _sample_body(cnt_ref, ecol_hbm, elog_hbm, out_ref, ec_s, el_s, sem0, sem1,
                 *, log2n):
    pid = pl.program_id(0)
    row = (pid * 1024
           + lax.broadcasted_iota(jnp.int32, (8, 128), 0) * 128
           + lax.broadcasted_iota(jnp.int32, (8, 128), 1))
    rmask = (1 << (32 - log2n)) - 1
    hi = (row >> (32 - log2n)).astype(jnp.uint32)
    lo_row = (row & rmask).astype(jnp.uint32) << np.uint32(log2n)
    cnt = cnt_ref[0]
    best0 = jnp.full((8, 128), _NEG, jnp.float32)
    bidx0 = jnp.zeros((8, 128), jnp.int32)

    def chunk_body(k, carry):
        best, bidx = carry
        off = k * _CHUNK
        c0 = pltpu.make_async_copy(ecol_hbm.at[pl.ds(off, _CHUNK)], ec_s, sem0)
        c1 = pltpu.make_async_copy(elog_hbm.at[pl.ds(off, _CHUNK)], el_s, sem1)
        c0.start()
        c1.start()
        c0.wait()
        c1.wait()
        m = jnp.minimum(cnt - off, _CHUNK)

        def cbody(j, carry2):
            best2, bidx2 = carry2
            ec = ec_s[j]
            el = el_s[j]
            lo = lo_row | ec.astype(jnp.uint32)
            g = _gumbel_from_bits(_threefry_bits(hi, lo))
            score = el + g
            upd = score > best2
            return jnp.where(upd, score, best2), jnp.where(upd, ec, bidx2)

        return lax.fori_loop(0, m, cbody, (best, bidx))

    nch = (cnt + _CHUNK - 1) // _CHUNK
    best, bidx = lax.fori_loop(0, nch, chunk_body, (best0, bidx0))
    out_ref[...] = bidx


def _sample(cnt, ecols, elog, n, log2n, interpret=False):
    grid = (n // 1024,)
    out = pl.pallas_call(
        functools.partial(_sample_body, log2n=log2n),
        grid=grid,
        in_specs=[
            pl.BlockSpec(memory_space=pltpu.SMEM),
            pl.BlockSpec(memory_space=pl.ANY),
            pl.BlockSpec(memory_space=pl.ANY),
        ],
        out_specs=pl.BlockSpec((8, 128), lambda i: (i, 0)),
        out_shape=jax.ShapeDtypeStruct((n // 128, 128), jnp.int32),
        scratch_shapes=[
            pltpu.SMEM((_CHUNK,), jnp.int32),
            pltpu.SMEM((_CHUNK,), jnp.float32),
            pltpu.SemaphoreType.DMA,
            pltpu.SemaphoreType.DMA,
        ],
        interpret=interpret,
    )(cnt, ecols, elog)
    return out.reshape(n)


def _make_sc_gather(n, d):
    mesh = plsc.VectorSubcoreMesh(core_axis_name="c", subcore_axis_name="s")
    nw = 32
    per_w = n // nw
    ch = 128
    nch = per_w // ch

    @functools.partial(
        pl.kernel,
        mesh=mesh,
        compiler_params=pltpu.CompilerParams(use_tc_tiling_on_sc=False),
        out_type=jax.ShapeDtypeStruct((n, d), jnp.float32),
        scratch_types=[
            pltpu.VMEM((ch,), jnp.int32),
            pltpu.VMEM((ch, d), jnp.float32),
            pltpu.SemaphoreType.DMA,
        ],
    )
    def gk(table_hbm, idx_hbm, out_hbm, idx_v, rows_v, sem):
        wid = lax.axis_index("s") * 2 + lax.axis_index("c")
        base = wid * per_w

        def body(i, carry):
            off = base + i * ch
            pltpu.sync_copy(idx_hbm.at[pl.ds(off, ch)], idx_v)
            pltpu.async_copy(table_hbm.at[idx_v], rows_v, sem).wait()
            pltpu.sync_copy(rows_v, out_hbm.at[pl.ds(off, ch)])
            return carry

        lax.fori_loop(0, nch, body, 0)

    return gk


def kernel(states, prior_states, sigma, onsets, init_ss, t_obs, s_obs,
           noise_scale, correct_prior, correct_lik, forget_lik):
    f32 = jnp.float32
    sig = jnp.maximum(sigma, _EPS)
    noise = jax.random.normal(jax.random.key(42), (_N, _D), f32)
    s2 = (sig * noise_scale[0]).reshape(_D, 1)
    alpha = forget_lik[0]
    idxb = jnp.arange(_B, dtype=f32) - (_B - 1)
    wts = jnp.exp(alpha * idxb)
    wts = wts / jnp.sum(wts)
    inv_var = 1.0 / (sig ** 2)
    coef = (correct_prior * np.float32(-0.5) * inv_var).reshape(_D, 1)

    ns_t, log_w = _forward(correct_lik.astype(f32), t_obs, s_obs, wts,
                           states.T, prior_states.T, noise.T, onsets, init_ss,
                           s2, coef, _N)
    new_states = ns_t.T
    new_weights = _softmax(log_w, _N)

    lmax = jnp.max(log_w)
    mask = log_w >= (lmax - np.float32(_WINDOW))
    cnt = jnp.sum(mask.astype(jnp.int32)).reshape(1)
    ecols = jnp.nonzero(mask, size=_N, fill_value=0)[0].astype(jnp.int32)
    elog = jnp.take(log_w, ecols)

    ridx = _sample(cnt, ecols, elog, _N, _LOG2N)
    resampled = _make_sc_gather(_N, _D)(new_states, ridx)
    return new_states, new_weights, resampled


# speedup vs baseline: 23.5345x; 4.0701x over previous
import functools

import jax
import jax.numpy as jnp
import numpy as np
from jax import lax
from jax.experimental import pallas as pl
from jax.experimental.pallas import tpu as pltpu
from jax.experimental.pallas import tpu_sc as plsc

_N = 262144
_LOG2N = 18
_D = 16
_B = 100
_EPS = 1e-6
_HALF_LOG_2PI = 0.9189385332046727
_WINDOW = 21.5
_TINY = float(np.finfo(np.float32).tiny)
_NEG = np.float32(-1e38)


def _threefry_bits(hi, lo):
    ks0 = np.uint32(0)
    ks1 = np.uint32(7)
    ks2 = np.uint32(0 ^ 7 ^ 0x1BD11BDA)
    x0 = hi + ks0
    x1 = lo + ks1
    ks = (ks0, ks1, ks2)
    rots = ((13, 15, 26, 6), (17, 29, 16, 24))
    for i in range(5):
        for r in rots[i % 2]:
            x0 = x0 + x1
            x1 = (x1 << np.uint32(r)) | (x1 >> np.uint32(32 - r))
            x1 = x1 ^ x0
        x0 = x0 + ks[(i + 1) % 3]
        x1 = x1 + ks[(i + 2) % 3] + np.uint32(i + 1)
    return x0 ^ x1


def _gumbel_from_bits(bits):
    fb = (bits >> np.uint32(9)) | np.uint32(0x3F800000)
    u0 = lax.bitcast_convert_type(fb, jnp.float32) - np.float32(1.0)
    u = jnp.maximum(np.float32(_TINY), u0 + np.float32(_TINY))
    return -jnp.log(-jnp.log(u))


def _softplus(x):
    return jnp.maximum(x, 0.0) + jnp.log1p(jnp.exp(-jnp.abs(x)))


def _fwd_body(par_ref, tobs_ref, sobs_ref, wts_ref, st_ref, pr_ref, nz_ref,
              on_ref, is_ref, s2_ref, coef_ref, ns_ref, lw_ref):
    ns = st_ref[...] + nz_ref[...] * s2_ref[...]
    ns_ref[...] = ns
    rate = _softplus(ns[0]) + np.float32(_EPS)
    disp = _softplus(ns[1]) + np.float32(_EPS)
    inv_r = 1.0 / rate
    inv_d = 1.0 / disp
    ldc = jnp.log(disp) + np.float32(_HALF_LOG_2PI)
    on = on_ref[...]
    iss = is_ref[...]
    cb = on.shape[0]

    def bbody(b, acc):
        tb = tobs_ref[b]
        sb = sobs_ref[b]
        wb = wts_ref[b]
        z = ((tb - on) - (sb - iss) * inv_r) * inv_d
        return acc + wb * (np.float32(-0.5) * (z * z) - ldc)

    acc = lax.fori_loop(0, _B, bbody, jnp.zeros((cb,), jnp.float32))
    diff = ns - pr_ref[...]
    accp = jnp.sum(coef_ref[...] * (diff * diff), axis=0)
    lw_ref[...] = par_ref[0] * acc + accp


def _forward(par, t_obs, s_obs, wts, st_t, pr_t, nz_t, onsets, init_ss, s2,
             coef, n, interpret=False):
    cb = 8192
    grid = (n // cb,)
    return pl.pallas_call(
        _fwd_body,
        grid=grid,
        in_specs=[
            pl.BlockSpec(memory_space=pltpu.SMEM),
            pl.BlockSpec(memory_space=pltpu.SMEM),
            pl.BlockSpec(memory_space=pltpu.SMEM),
            pl.BlockSpec(memory_space=pltpu.SMEM),
            pl.BlockSpec((_D, cb), lambda i: (0, i)),
            pl.BlockSpec((_D, cb), lambda i: (0, i)),
            pl.BlockSpec((_D, cb), lambda i: (0, i)),
            pl.BlockSpec((cb,), lambda i: (i,)),
            pl.BlockSpec((cb,), lambda i: (i,)),
            pl.BlockSpec((_D, 1), lambda i: (0, 0)),
            pl.BlockSpec((_D, 1), lambda i: (0, 0)),
        ],
        out_specs=[
            pl.BlockSpec((_D, cb), lambda i: (0, i)),
            pl.BlockSpec((cb,), lambda i: (i,)),
        ],
        out_shape=[
            jax.ShapeDtypeStruct((_D, n), jnp.float32),
            jax.ShapeDtypeStruct((n,), jnp.float32),
        ],
        interpret=interpret,
    )(par, t_obs, s_obs, wts, st_t, pr_t, nz_t, onsets, init_ss, s2, coef)


def _softmax_body(lw_ref, w_ref):
    lw = lw_ref[...]
    e = jnp.exp(lw - jnp.max(lw))
    w_ref[...] = e / jnp.sum(e)


def _softmax(log_w, n, interpret=False):
    return pl.pallas_call(
        _softmax_body,
        out_shape=jax.ShapeDtypeStruct((n,), jnp.float32),
        interpret=interpret,
    )(log_w)


_CHUNK = 2048
_UNROLL = 8


def _sample_body(cnt_ref, ecol_hbm, elog_hbm, out_ref, ec_s, el_s, sem0, sem1,
                 *, log2n):
    pid = pl.program_id(0)
    row = (pid * 1024
           + lax.broadcasted_iota(jnp.int32, (8, 128), 0) * 128
           + lax.broadcasted_iota(jnp.int32, (8, 128), 1))
    rmask = (1 << (32 - log2n)) - 1
    hi = (row >> (32 - log2n)).astype(jnp.uint32)
    lo_row = (row & rmask).astype(jnp.uint32) << np.uint32(log2n)
    cnt = cnt_ref[0]
    best0 = jnp.full((8, 128), _NEG, jnp.float32)
    bidx0 = jnp.zeros((8, 128), jnp.int32)

    def chunk_body(k, carry):
        best, bidx = carry
        off = k * _CHUNK
        c0 = pltpu.make_async_copy(ecol_hbm.at[pl.ds(off, _CHUNK)], ec_s, sem0)
        c1 = pltpu.make_async_copy(elog_hbm.at[pl.ds(off, _CHUNK)], el_s, sem1)
        c0.start()
        c1.start()
        c0.wait()
        c1.wait()
        m = jnp.minimum(cnt - off, _CHUNK)

        def cbody(t, carry2):
            best2, bidx2 = carry2
            base = t * _UNROLL
            scores = []
            ecs = []
            for u in range(_UNROLL):
                ec = ec_s[base + u]
                el = el_s[base + u]
                lo = lo_row | ec.astype(jnp.uint32)
                g = _gumbel_from_bits(_threefry_bits(hi, lo))
                scores.append(el + g)
                ecs.append(ec)
            for u in range(_UNROLL):
                upd = scores[u] > best2
                best2 = jnp.where(upd, scores[u], best2)
                bidx2 = jnp.where(upd, ecs[u], bidx2)
            return best2, bidx2

        nt = (m + _UNROLL - 1) // _UNROLL
        return lax.fori_loop(0, nt, cbody, (best, bidx))

    nch = (cnt + _CHUNK - 1) // _CHUNK
    best, bidx = lax.fori_loop(0, nch, chunk_body, (best0, bidx0))
    out_ref[...] = bidx


def _sample(cnt, ecols, elog, n, log2n, interpret=False):
    grid = (n // 1024,)
    out = pl.pallas_call(
        functools.partial(_sample_body, log2n=log2n),
        grid=grid,
        in_specs=[
            pl.BlockSpec(memory_space=pltpu.SMEM),
            pl.BlockSpec(memory_space=pl.ANY),
            pl.BlockSpec(memory_space=pl.ANY),
        ],
        out_specs=pl.BlockSpec((8, 128), lambda i: (i, 0)),
        out_shape=jax.ShapeDtypeStruct((n // 128, 128), jnp.int32),
        scratch_shapes=[
            pltpu.SMEM((_CHUNK,), jnp.int32),
            pltpu.SMEM((_CHUNK,), jnp.float32),
            pltpu.SemaphoreType.DMA,
            pltpu.SemaphoreType.DMA,
        ],
        interpret=interpret,
    )(cnt, ecols, elog)
    return out.reshape(n)


def _make_sc_gather(n, d):
    mesh = plsc.VectorSubcoreMesh(core_axis_name="c", subcore_axis_name="s")
    nw = 32
    per_w = n // nw
    ch = 128
    nch = per_w // ch

    @functools.partial(
        pl.kernel,
        mesh=mesh,
        compiler_params=pltpu.CompilerParams(use_tc_tiling_on_sc=False),
        out_type=jax.ShapeDtypeStruct((n, d), jnp.float32),
        scratch_types=[
            pltpu.VMEM((ch,), jnp.int32),
            pltpu.VMEM((ch, d), jnp.float32),
            pltpu.SemaphoreType.DMA,
        ],
    )
    def gk(table_hbm, idx_hbm, out_hbm, idx_v, rows_v, sem):
        wid = lax.axis_index("s") * 2 + lax.axis_index("c")
        base = wid * per_w

        def body(i, carry):
            off = base + i * ch
            pltpu.sync_copy(idx_hbm.at[pl.ds(off, ch)], idx_v)
            pltpu.async_copy(table_hbm.at[idx_v], rows_v, sem).wait()
            pltpu.sync_copy(rows_v, out_hbm.at[pl.ds(off, ch)])
            return carry

        lax.fori_loop(0, nch, body, 0)

    return gk


def kernel(states, prior_states, sigma, onsets, init_ss, t_obs, s_obs,
           noise_scale, correct_prior, correct_lik, forget_lik):
    f32 = jnp.float32
    sig = jnp.maximum(sigma, _EPS)
    noise = jax.random.normal(jax.random.key(42), (_N, _D), f32)
    s2 = (sig * noise_scale[0]).reshape(_D, 1)
    alpha = forget_lik[0]
    idxb = jnp.arange(_B, dtype=f32) - (_B - 1)
    wts = jnp.exp(alpha * idxb)
    wts = wts / jnp.sum(wts)
    inv_var = 1.0 / (sig ** 2)
    coef = (correct_prior * np.float32(-0.5) * inv_var).reshape(_D, 1)

    ns_t, log_w = _forward(correct_lik.astype(f32), t_obs, s_obs, wts,
                           states.T, prior_states.T, noise.T, onsets, init_ss,
                           s2, coef, _N)
    new_states = ns_t.T
    new_weights = _softmax(log_w, _N)

    lmax = jnp.max(log_w)
    mask = log_w >= (lmax - np.float32(_WINDOW))
    cnt = jnp.sum(mask.astype(jnp.int32)).reshape(1)
    ecols = jnp.nonzero(mask, size=_N, fill_value=0)[0].astype(jnp.int32)
    elog = jnp.where(jnp.arange(_N) < cnt[0], jnp.take(log_w, ecols), _NEG)

    ridx = _sample(cnt, ecols, elog, _N, _LOG2N)
    resampled = _make_sc_gather(_N, _D)(new_states, ridx)
    return new_states, new_weights, resampled


# speedup vs baseline: 26.6580x; 1.1327x over previous
import functools

import jax
import jax.numpy as jnp
import numpy as np
from jax import lax
from jax.experimental import pallas as pl
from jax.experimental.pallas import tpu as pltpu
from jax.experimental.pallas import tpu_sc as plsc

_N = 262144
_LOG2N = 18
_D = 16
_B = 100
_EPS = 1e-6
_HALF_LOG_2PI = 0.9189385332046727
_WINDOW = 21.5
_TINY = float(np.finfo(np.float32).tiny)
_NEG = np.float32(-1e38)


def _threefry_bits(hi, lo):
    ks0 = np.uint32(0)
    ks1 = np.uint32(7)
    ks2 = np.uint32(0 ^ 7 ^ 0x1BD11BDA)
    x0 = hi + ks0
    x1 = lo + ks1
    ks = (ks0, ks1, ks2)
    rots = ((13, 15, 26, 6), (17, 29, 16, 24))
    for i in range(5):
        for r in rots[i % 2]:
            x0 = x0 + x1
            x1 = (x1 << np.uint32(r)) | (x1 >> np.uint32(32 - r))
            x1 = x1 ^ x0
        x0 = x0 + ks[(i + 1) % 3]
        x1 = x1 + ks[(i + 2) % 3] + np.uint32(i + 1)
    return x0 ^ x1


def _gumbel_from_bits(bits):
    fb = (bits >> np.uint32(9)) | np.uint32(0x3F800000)
    u0 = lax.bitcast_convert_type(fb, jnp.float32) - np.float32(1.0)
    u = jnp.maximum(np.float32(_TINY), u0 + np.float32(_TINY))
    return -jnp.log(-jnp.log(u))


def _softplus(x):
    return jnp.maximum(x, 0.0) + jnp.log1p(jnp.exp(-jnp.abs(x)))


def _fwd_body(par_ref, tobs_ref, sobs_ref, wts_ref, st_ref, pr_ref, nz_ref,
              on_ref, is_ref, s2_ref, coef_ref, ns_ref, lw_ref):
    ns = st_ref[...] + nz_ref[...] * s2_ref[...]
    ns_ref[...] = ns
    rate = _softplus(ns[0]) + np.float32(_EPS)
    disp = _softplus(ns[1]) + np.float32(_EPS)
    inv_r = 1.0 / rate
    inv_d = 1.0 / disp
    ldc = jnp.log(disp) + np.float32(_HALF_LOG_2PI)
    on = on_ref[...]
    iss = is_ref[...]
    cb = on.shape[0]

    def bbody(b, acc):
        tb = tobs_ref[b]
        sb = sobs_ref[b]
        wb = wts_ref[b]
        z = ((tb - on) - (sb - iss) * inv_r) * inv_d
        return acc + wb * (np.float32(-0.5) * (z * z) - ldc)

    acc = lax.fori_loop(0, _B, bbody, jnp.zeros((cb,), jnp.float32))
    diff = ns - pr_ref[...]
    accp = jnp.sum(coef_ref[...] * (diff * diff), axis=0)
    lw_ref[...] = par_ref[0] * acc + accp


def _forward(par, t_obs, s_obs, wts, st_t, pr_t, nz_t, onsets, init_ss, s2,
             coef, n, interpret=False):
    cb = 8192
    grid = (n // cb,)
    return pl.pallas_call(
        _fwd_body,
        grid=grid,
        in_specs=[
            pl.BlockSpec(memory_space=pltpu.SMEM),
            pl.BlockSpec(memory_space=pltpu.SMEM),
            pl.BlockSpec(memory_space=pltpu.SMEM),
            pl.BlockSpec(memory_space=pltpu.SMEM),
            pl.BlockSpec((_D, cb), lambda i: (0, i)),
            pl.BlockSpec((_D, cb), lambda i: (0, i)),
            pl.BlockSpec((_D, cb), lambda i: (0, i)),
            pl.BlockSpec((cb,), lambda i: (i,)),
            pl.BlockSpec((cb,), lambda i: (i,)),
            pl.BlockSpec((_D, 1), lambda i: (0, 0)),
            pl.BlockSpec((_D, 1), lambda i: (0, 0)),
        ],
        out_specs=[
            pl.BlockSpec((_D, cb), lambda i: (0, i)),
            pl.BlockSpec((cb,), lambda i: (i,)),
        ],
        out_shape=[
            jax.ShapeDtypeStruct((_D, n), jnp.float32),
            jax.ShapeDtypeStruct((n,), jnp.float32),
        ],
        compiler_params=pltpu.CompilerParams(
            dimension_semantics=("parallel",)),
        interpret=interpret,
    )(par, t_obs, s_obs, wts, st_t, pr_t, nz_t, onsets, init_ss, s2, coef)


def _softmax_body(lw_ref, w_ref):
    lw = lw_ref[...]
    e = jnp.exp(lw - jnp.max(lw))
    w_ref[...] = e / jnp.sum(e)


def _softmax(log_w, n, interpret=False):
    return pl.pallas_call(
        _softmax_body,
        out_shape=jax.ShapeDtypeStruct((n,), jnp.float32),
        interpret=interpret,
    )(log_w)


_CHUNK = 2048
_UNROLL = 16


def _sample_body(cnt_ref, ecol_hbm, elog_hbm, out_ref, ec_s, el_s, sem0, sem1,
                 *, log2n):
    pid = pl.program_id(0)
    row = (pid * 1024
           + lax.broadcasted_iota(jnp.int32, (8, 128), 0) * 128
           + lax.broadcasted_iota(jnp.int32, (8, 128), 1))
    rmask = (1 << (32 - log2n)) - 1
    hi = (row >> (32 - log2n)).astype(jnp.uint32)
    lo_row = (row & rmask).astype(jnp.uint32) << np.uint32(log2n)
    cnt = cnt_ref[0]
    best0 = jnp.full((8, 128), _NEG, jnp.float32)
    bidx0 = jnp.zeros((8, 128), jnp.int32)

    def chunk_body(k, carry):
        best, bidx = carry
        off = k * _CHUNK
        c0 = pltpu.make_async_copy(ecol_hbm.at[pl.ds(off, _CHUNK)], ec_s, sem0)
        c1 = pltpu.make_async_copy(elog_hbm.at[pl.ds(off, _CHUNK)], el_s, sem1)
        c0.start()
        c1.start()
        c0.wait()
        c1.wait()
        m = jnp.minimum(cnt - off, _CHUNK)

        def cbody(t, carry2):
            best2, bidx2 = carry2
            base = t * _UNROLL
            scores = []
            ecs = []
            for u in range(_UNROLL):
                ec = ec_s[base + u]
                el = el_s[base + u]
                lo = lo_row | ec.astype(jnp.uint32)
                g = _gumbel_from_bits(_threefry_bits(hi, lo))
                scores.append(el + g)
                ecs.append(ec)
            for u in range(_UNROLL):
                upd = scores[u] > best2
                best2 = jnp.where(upd, scores[u], best2)
                bidx2 = jnp.where(upd, ecs[u], bidx2)
            return best2, bidx2

        nt = (m + _UNROLL - 1) // _UNROLL
        return lax.fori_loop(0, nt, cbody, (best, bidx))

    nch = (cnt + _CHUNK - 1) // _CHUNK
    best, bidx = lax.fori_loop(0, nch, chunk_body, (best0, bidx0))
    out_ref[...] = bidx


def _sample(cnt, ecols, elog, n, log2n, interpret=False):
    grid = (n // 1024,)
    out = pl.pallas_call(
        functools.partial(_sample_body, log2n=log2n),
        grid=grid,
        in_specs=[
            pl.BlockSpec(memory_space=pltpu.SMEM),
            pl.BlockSpec(memory_space=pl.ANY),
            pl.BlockSpec(memory_space=pl.ANY),
        ],
        out_specs=pl.BlockSpec((8, 128), lambda i: (i, 0)),
        out_shape=jax.ShapeDtypeStruct((n // 128, 128), jnp.int32),
        scratch_shapes=[
            pltpu.SMEM((_CHUNK,), jnp.int32),
            pltpu.SMEM((_CHUNK,), jnp.float32),
            pltpu.SemaphoreType.DMA,
            pltpu.SemaphoreType.DMA,
        ],
        compiler_params=pltpu.CompilerParams(
            dimension_semantics=("parallel",)),
        interpret=interpret,
    )(cnt, ecols, elog)
    return out.reshape(n)


def _make_sc_gather(n, d):
    mesh = plsc.VectorSubcoreMesh(core_axis_name="c", subcore_axis_name="s")
    nw = 32
    per_w = n // nw
    ch = 128
    nch = per_w // ch

    @functools.partial(
        pl.kernel,
        mesh=mesh,
        compiler_params=pltpu.CompilerParams(use_tc_tiling_on_sc=False),
        out_type=jax.ShapeDtypeStruct((n, d), jnp.float32),
        scratch_types=[
            pltpu.VMEM((ch,), jnp.int32),
            pltpu.VMEM((ch, d), jnp.float32),
            pltpu.SemaphoreType.DMA,
        ],
    )
    def gk(table_hbm, idx_hbm, out_hbm, idx_v, rows_v, sem):
        wid = lax.axis_index("s") * 2 + lax.axis_index("c")
        base = wid * per_w

        def body(i, carry):
            off = base + i * ch
            pltpu.sync_copy(idx_hbm.at[pl.ds(off, ch)], idx_v)
            pltpu.async_copy(table_hbm.at[idx_v], rows_v, sem).wait()
            pltpu.sync_copy(rows_v, out_hbm.at[pl.ds(off, ch)])
            return carry

        lax.fori_loop(0, nch, body, 0)

    return gk


def kernel(states, prior_states, sigma, onsets, init_ss, t_obs, s_obs,
           noise_scale, correct_prior, correct_lik, forget_lik):
    f32 = jnp.float32
    sig = jnp.maximum(sigma, _EPS)
    noise = jax.random.normal(jax.random.key(42), (_N, _D), f32)
    s2 = (sig * noise_scale[0]).reshape(_D, 1)
    alpha = forget_lik[0]
    idxb = jnp.arange(_B, dtype=f32) - (_B - 1)
    wts = jnp.exp(alpha * idxb)
    wts = wts / jnp.sum(wts)
    inv_var = 1.0 / (sig ** 2)
    coef = (correct_prior * np.float32(-0.5) * inv_var).reshape(_D, 1)

    ns_t, log_w = _forward(correct_lik.astype(f32), t_obs, s_obs, wts,
                           states.T, prior_states.T, noise.T, onsets, init_ss,
                           s2, coef, _N)
    new_states = ns_t.T
    new_weights = _softmax(log_w, _N)

    lmax = jnp.max(log_w)
    mask = log_w >= (lmax - np.float32(_WINDOW))
    cnt = jnp.sum(mask.astype(jnp.int32)).reshape(1)
    ecols = jnp.nonzero(mask, size=_N, fill_value=0)[0].astype(jnp.int32)
    elog = jnp.where(jnp.arange(_N) < cnt[0], jnp.take(log_w, ecols), _NEG)

    ridx = _sample(cnt, ecols, elog, _N, _LOG2N)
    resampled = _make_sc_gather(_N, _D)(new_states, ridx)
    return new_states, new_weights, resampled


# speedup vs baseline: 44.8859x; 1.6838x over previous
import functools

import jax
import jax.numpy as jnp
import numpy as np
from jax import lax
from jax.experimental import pallas as pl
from jax.experimental.pallas import tpu as pltpu
from jax.experimental.pallas import tpu_sc as plsc

_N = 262144
_LOG2N = 18
_D = 16
_B = 100
_EPS = 1e-6
_HALF_LOG_2PI = 0.9189385332046727
_WINDOW = 21.5
_TINY = float(np.finfo(np.float32).tiny)
_NEG = np.float32(-1e38)


def _threefry_bits(hi, lo):
    ks0 = np.uint32(0)
    ks1 = np.uint32(7)
    ks2 = np.uint32(0 ^ 7 ^ 0x1BD11BDA)
    x0 = hi + ks0
    x1 = lo + ks1
    ks = (ks0, ks1, ks2)
    rots = ((13, 15, 26, 6), (17, 29, 16, 24))
    for i in range(5):
        for r in rots[i % 2]:
            x0 = x0 + x1
            x1 = (x1 << np.uint32(r)) | (x1 >> np.uint32(32 - r))
            x1 = x1 ^ x0
        x0 = x0 + ks[(i + 1) % 3]
        x1 = x1 + ks[(i + 2) % 3] + np.uint32(i + 1)
    return x0 ^ x1


def _gumbel_from_bits(bits):
    fb = (bits >> np.uint32(9)) | np.uint32(0x3F800000)
    u0 = lax.bitcast_convert_type(fb, jnp.float32) - np.float32(1.0)
    u = jnp.maximum(np.float32(_TINY), u0 + np.float32(_TINY))
    return -jnp.log(-jnp.log(u))


def _softplus(x):
    return jnp.maximum(x, 0.0) + jnp.log1p(jnp.exp(-jnp.abs(x)))


def _fwd_body(par_ref, tobs_ref, sobs_ref, wts_ref, st_ref, pr_ref, nz_ref,
              on_ref, is_ref, s2_ref, coef_ref, ns_ref, lw_ref):
    ns = st_ref[...] + nz_ref[...] * s2_ref[...]
    ns_ref[...] = ns
    rate = _softplus(ns[0]) + np.float32(_EPS)
    disp = _softplus(ns[1]) + np.float32(_EPS)
    inv_r = 1.0 / rate
    inv_d = 1.0 / disp
    ldc = jnp.log(disp) + np.float32(_HALF_LOG_2PI)
    on = on_ref[...]
    iss = is_ref[...]
    cb = on.shape[0]

    def bbody(b, acc):
        tb = tobs_ref[b]
        sb = sobs_ref[b]
        wb = wts_ref[b]
        z = ((tb - on) - (sb - iss) * inv_r) * inv_d
        return acc + wb * (np.float32(-0.5) * (z * z) - ldc)

    acc = lax.fori_loop(0, _B, bbody, jnp.zeros((cb,), jnp.float32))
    diff = ns - pr_ref[...]
    accp = jnp.sum(coef_ref[...] * (diff * diff), axis=0)
    lw_ref[...] = par_ref[0] * acc + accp


def _forward(par, t_obs, s_obs, wts, st_t, pr_t, nz_t, onsets, init_ss, s2,
             coef, n, interpret=False):
    cb = 8192
    grid = (n // cb,)
    return pl.pallas_call(
        _fwd_body,
        grid=grid,
        in_specs=[
            pl.BlockSpec(memory_space=pltpu.SMEM),
            pl.BlockSpec(memory_space=pltpu.SMEM),
            pl.BlockSpec(memory_space=pltpu.SMEM),
            pl.BlockSpec(memory_space=pltpu.SMEM),
            pl.BlockSpec((_D, cb), lambda i: (0, i)),
            pl.BlockSpec((_D, cb), lambda i: (0, i)),
            pl.BlockSpec((_D, cb), lambda i: (0, i)),
            pl.BlockSpec((cb,), lambda i: (i,)),
            pl.BlockSpec((cb,), lambda i: (i,)),
            pl.BlockSpec((_D, 1), lambda i: (0, 0)),
            pl.BlockSpec((_D, 1), lambda i: (0, 0)),
        ],
        out_specs=[
            pl.BlockSpec((_D, cb), lambda i: (0, i)),
            pl.BlockSpec((cb,), lambda i: (i,)),
        ],
        out_shape=[
            jax.ShapeDtypeStruct((_D, n), jnp.float32),
            jax.ShapeDtypeStruct((n,), jnp.float32),
        ],
        compiler_params=pltpu.CompilerParams(
            dimension_semantics=("parallel",)),
        interpret=interpret,
    )(par, t_obs, s_obs, wts, st_t, pr_t, nz_t, onsets, init_ss, s2, coef)


def _softmax_body(lw_ref, w_ref):
    lw = lw_ref[...]
    e = jnp.exp(lw - jnp.max(lw))
    w_ref[...] = e / jnp.sum(e)


def _softmax(log_w, n, interpret=False):
    return pl.pallas_call(
        _softmax_body,
        out_shape=jax.ShapeDtypeStruct((n,), jnp.float32),
        interpret=interpret,
    )(log_w)


_CHUNK = 2048
_UNROLL = 16


def _sample_body(cnt_ref, ecol_hbm, elog_hbm, out_ref, ec_s, el_s, sem0, sem1,
                 *, log2n):
    pid = pl.program_id(0)
    row = (pid * 1024
           + lax.broadcasted_iota(jnp.int32, (8, 128), 0) * 128
           + lax.broadcasted_iota(jnp.int32, (8, 128), 1))
    rmask = (1 << (32 - log2n)) - 1
    hi = (row >> (32 - log2n)).astype(jnp.uint32)
    lo_row = (row & rmask).astype(jnp.uint32) << np.uint32(log2n)
    cnt = cnt_ref[0]
    best0 = jnp.full((8, 128), _NEG, jnp.float32)
    bidx0 = jnp.zeros((8, 128), jnp.int32)

    def chunk_body(k, carry):
        best_in, bidx_in, done_in = carry

        def run(args):
            best, bidx = args
            off = k * _CHUNK
            c0 = pltpu.make_async_copy(
                ecol_hbm.at[pl.ds(off, _CHUNK)], ec_s, sem0)
            c1 = pltpu.make_async_copy(
                elog_hbm.at[pl.ds(off, _CHUNK)], el_s, sem1)
            c0.start()
            c1.start()
            c0.wait()
            c1.wait()
            m = jnp.minimum(cnt - off, _CHUNK)
            nt = (m + _UNROLL - 1) // _UNROLL

            def wcond(c):
                t, best2, _ = c
                thr = jnp.min(best2) - np.float32(16.0)
                return jnp.logical_and(t < nt, el_s[t * _UNROLL] > thr)

            def wbody(c):
                t, best2, bidx2 = c
                base = t * _UNROLL
                scores = []
                ecs = []
                for u in range(_UNROLL):
                    ec = ec_s[base + u]
                    el = el_s[base + u]
                    lo = lo_row | ec.astype(jnp.uint32)
                    g = _gumbel_from_bits(_threefry_bits(hi, lo))
                    scores.append(el + g)
                    ecs.append(ec)
                for u in range(_UNROLL):
                    upd = scores[u] > best2
                    best2 = jnp.where(upd, scores[u], best2)
                    bidx2 = jnp.where(upd, ecs[u], bidx2)
                return t + 1, best2, bidx2

            t, best, bidx = lax.while_loop(
                wcond, wbody, (jnp.int32(0), best, bidx))
            return best, bidx, t < nt

        def skip(args):
            best, bidx = args
            return best, bidx, jnp.bool_(True)

        return lax.cond(done_in, skip, run, (best_in, bidx_in))

    nch = (cnt + _CHUNK - 1) // _CHUNK
    best, bidx, _ = lax.fori_loop(
        0, nch, chunk_body, (best0, bidx0, jnp.bool_(False)))
    out_ref[...] = bidx


def _sample(cnt, ecols, elog, n, log2n, interpret=False):
    grid = (n // 1024,)
    out = pl.pallas_call(
        functools.partial(_sample_body, log2n=log2n),
        grid=grid,
        in_specs=[
            pl.BlockSpec(memory_space=pltpu.SMEM),
            pl.BlockSpec(memory_space=pl.ANY),
            pl.BlockSpec(memory_space=pl.ANY),
        ],
        out_specs=pl.BlockSpec((8, 128), lambda i: (i, 0)),
        out_shape=jax.ShapeDtypeStruct((n // 128, 128), jnp.int32),
        scratch_shapes=[
            pltpu.SMEM((_CHUNK,), jnp.int32),
            pltpu.SMEM((_CHUNK,), jnp.float32),
            pltpu.SemaphoreType.DMA,
            pltpu.SemaphoreType.DMA,
        ],
        compiler_params=pltpu.CompilerParams(
            dimension_semantics=("parallel",)),
        interpret=interpret,
    )(cnt, ecols, elog)
    return out.reshape(n)


def _make_sc_gather(n, d):
    mesh = plsc.VectorSubcoreMesh(core_axis_name="c", subcore_axis_name="s")
    nw = 32
    per_w = n // nw
    ch = 128
    nch = per_w // ch

    @functools.partial(
        pl.kernel,
        mesh=mesh,
        compiler_params=pltpu.CompilerParams(use_tc_tiling_on_sc=False),
        out_type=jax.ShapeDtypeStruct((n, d), jnp.float32),
        scratch_types=[
            pltpu.VMEM((ch,), jnp.int32),
            pltpu.VMEM((ch, d), jnp.float32),
            pltpu.SemaphoreType.DMA,
        ],
    )
    def gk(table_hbm, idx_hbm, out_hbm, idx_v, rows_v, sem):
        wid = lax.axis_index("s") * 2 + lax.axis_index("c")
        base = wid * per_w

        def body(i, carry):
            off = base + i * ch
            pltpu.sync_copy(idx_hbm.at[pl.ds(off, ch)], idx_v)
            pltpu.async_copy(table_hbm.at[idx_v], rows_v, sem).wait()
            pltpu.sync_copy(rows_v, out_hbm.at[pl.ds(off, ch)])
            return carry

        lax.fori_loop(0, nch, body, 0)

    return gk


def kernel(states, prior_states, sigma, onsets, init_ss, t_obs, s_obs,
           noise_scale, correct_prior, correct_lik, forget_lik):
    f32 = jnp.float32
    sig = jnp.maximum(sigma, _EPS)
    noise = jax.random.normal(jax.random.key(42), (_N, _D), f32)
    s2 = (sig * noise_scale[0]).reshape(_D, 1)
    alpha = forget_lik[0]
    idxb = jnp.arange(_B, dtype=f32) - (_B - 1)
    wts = jnp.exp(alpha * idxb)
    wts = wts / jnp.sum(wts)
    inv_var = 1.0 / (sig ** 2)
    coef = (correct_prior * np.float32(-0.5) * inv_var).reshape(_D, 1)

    ns_t, log_w = _forward(correct_lik.astype(f32), t_obs, s_obs, wts,
                           states.T, prior_states.T, noise.T, onsets, init_ss,
                           s2, coef, _N)
    new_states = ns_t.T
    new_weights = _softmax(log_w, _N)

    lmax = jnp.max(log_w)
    mask = log_w >= (lmax - np.float32(_WINDOW))
    cnt = jnp.sum(mask.astype(jnp.int32)).reshape(1)
    neg = jnp.where(mask, -log_w, jnp.float32(np.inf))
    skey, ecols = lax.sort_key_val(neg, jnp.arange(_N, dtype=jnp.int32))
    elog = jnp.where(jnp.arange(_N) < cnt[0], -skey, _NEG)

    ridx = _sample(cnt, ecols, elog, _N, _LOG2N)
    resampled = _make_sc_gather(_N, _D)(new_states, ridx)
    return new_states, new_weights, resampled
